# baseline (speedup 1.0000x reference)
import functools

import numpy as np
import jax
import jax.numpy as jnp
from jax import lax
from jax.experimental import pallas as pl
from jax.experimental.pallas import tpu as pltpu
from jax.experimental.pallas import tpu_sc as plsc

_EPS = 1e-5
_NC = 2
_NS = 16
_NW = _NC * _NS
_CH = 128


_KJ = 13


def _sc_gather(table16, idx1d):
    e = idx1d.shape[0]
    nch = e // _CH
    assert nch * _CH == e
    per = nch // _NW
    rem = nch - per * _NW
    ng = per // _KJ
    assert ng * _KJ == per, (nch, per)
    rows_n = _KJ * _CH
    mesh = plsc.VectorSubcoreMesh(core_axis_name="c", subcore_axis_name="s")

    @functools.partial(
        pl.kernel,
        out_type=jax.ShapeDtypeStruct((e, 16), jnp.float32),
        mesh=mesh,
        compiler_params=pltpu.CompilerParams(use_tc_tiling_on_sc=False),
        scratch_types=[
            pltpu.VMEM((rows_n,), jnp.int32),
            pltpu.VMEM((rows_n, 16), jnp.float32),
            pltpu.VMEM((_CH,), jnp.int32),
            pltpu.VMEM((_CH, 16), jnp.float32),
            pltpu.SemaphoreType.DMA,
        ],
    )
    def k(table_hbm, idx_hbm, out_hbm, idx_b, rows_v, idx_e, rows_e, sem):
        wid = lax.axis_index("s") * _NC + lax.axis_index("c")
        start = wid * per

        def group(g, carry):
            gc = start + g * _KJ
            pltpu.sync_copy(idx_hbm.at[pl.ds(gc * _CH, rows_n)], idx_b)
            descs = [
                pltpu.async_copy(table_hbm.at[idx_b.at[pl.ds(j * _CH, _CH)]],
                                 rows_v.at[pl.ds(j * _CH, _CH)], sem)
                for j in range(_KJ)
            ]
            for d in descs:
                d.wait()
            pltpu.sync_copy(rows_v, out_hbm.at[pl.ds(gc * _CH, rows_n)])
            return carry

        lax.fori_loop(0, ng, group, 0)

        @pl.when(wid < rem)
        def _():
            ec = _NW * per + wid
            pltpu.sync_copy(idx_hbm.at[pl.ds(ec * _CH, _CH)], idx_e)
            pltpu.async_copy(table_hbm.at[idx_e], rows_e, sem).wait()
            pltpu.sync_copy(rows_e, out_hbm.at[pl.ds(ec * _CH, _CH)])

    return k(table16, idx1d)


def _sc_scatter(msg16, idx1d, zeros16):
    e = idx1d.shape[0]
    n = zeros16.shape[0]
    nch = e // _CH
    per = nch // _NW
    rem = nch - per * _NW
    ng = per // _KJ
    assert ng * _KJ == per, (nch, per)
    rows_n = _KJ * _CH
    seg = n // _NS
    assert seg * _NS == n
    mesh = plsc.VectorSubcoreMesh(core_axis_name="c", subcore_axis_name="s")

    @functools.partial(
        pl.kernel,
        out_type=jax.ShapeDtypeStruct((_NC, n, 16), jnp.float32),
        mesh=mesh,
        compiler_params=pltpu.CompilerParams(use_tc_tiling_on_sc=False),
        scratch_types=[
            pltpu.VMEM((rows_n,), jnp.int32),
            pltpu.VMEM((rows_n, 16), jnp.float32),
            pltpu.VMEM((_CH,), jnp.int32),
            pltpu.VMEM((_CH, 16), jnp.float32),
            pltpu.VMEM_SHARED((n, 16), jnp.float32),
            pltpu.SemaphoreType.DMA,
        ],
    )
    def k(msg_hbm, idx_hbm, zeros_hbm, out_hbm, idx_b, rows_v, idx_e, rows_e,
          acc, sem):
        c = lax.axis_index("c")
        s = lax.axis_index("s")
        wid = s * _NC + c
        start = wid * per
        pltpu.sync_copy(zeros_hbm.at[pl.ds(s * seg, seg)],
                        acc.at[pl.ds(s * seg, seg)])
        plsc.subcore_barrier()

        def group(g, carry):
            gc = start + g * _KJ
            pltpu.sync_copy(idx_hbm.at[pl.ds(gc * _CH, rows_n)], idx_b)
            pltpu.sync_copy(msg_hbm.at[pl.ds(gc * _CH, rows_n)], rows_v)
            descs = [
                pltpu.async_copy(rows_v.at[pl.ds(j * _CH, _CH)],
                                 acc.at[idx_b.at[pl.ds(j * _CH, _CH)]],
                                 sem, add=True)
                for j in range(_KJ)
            ]
            for d in descs:
                d.wait()
            return carry

        lax.fori_loop(0, ng, group, 0)

        @pl.when(wid < rem)
        def _():
            ec = _NW * per + wid
            pltpu.sync_copy(idx_hbm.at[pl.ds(ec * _CH, _CH)], idx_e)
            pltpu.sync_copy(msg_hbm.at[pl.ds(ec * _CH, _CH)], rows_e)
            pltpu.sync_copy(rows_e, acc.at[idx_e], add=True)

        plsc.subcore_barrier()
        pltpu.sync_copy(acc.at[pl.ds(s * seg, seg)],
                        out_hbm.at[c, pl.ds(s * seg, seg)])

    return k(msg16, idx1d, zeros16)


def _msg_body(xs_ref, ea_ref, wnn_ref, bnn_ref, r_ref, s_ref, e8_ref, out_ref):
    m = jnp.tanh(jnp.dot(ea_ref[...], wnn_ref[...],
                         preferred_element_type=jnp.float32) + bnn_ref[...])
    xr = jnp.dot(xs_ref[...], r_ref[...], preferred_element_type=jnp.float32)
    msg = jnp.dot(xr * m, s_ref[...], preferred_element_type=jnp.float32)
    out_ref[...] = msg + e8_ref[...]


def _tc_msg(xs_w, ea_w, wnn_b, bnn_b, r_b, s_b, e8_b):
    e8r = xs_w.shape[0]
    te = 1000 if e8r % 1000 == 0 else e8r
    grid = e8r // te
    return pl.pallas_call(
        _msg_body,
        grid=(grid,),
        in_specs=[
            pl.BlockSpec((te, 128), lambda i: (i, 0)),
            pl.BlockSpec((te, 128), lambda i: (i, 0)),
            pl.BlockSpec((128, 512), lambda i: (0, 0)),
            pl.BlockSpec((1, 512), lambda i: (0, 0)),
            pl.BlockSpec((128, 512), lambda i: (0, 0)),
            pl.BlockSpec((512, 128), lambda i: (0, 0)),
            pl.BlockSpec((1, 128), lambda i: (0, 0)),
        ],
        out_specs=pl.BlockSpec((te, 128), lambda i: (i, 0)),
        out_shape=jax.ShapeDtypeStruct((e8r, 128), jnp.float32),
    )(xs_w, ea_w, wnn_b, bnn_b, r_b, s_b, e8_b)


def _tn_for(n):
    return 2000 if n % 2000 == 0 else n


def _finA_body(x_ref, p0_ref, p1_ref, wr_ref, br_ref, h_ref, st_ref, acc):
    i = pl.program_id(0)
    p = p0_ref[...] + p1_ref[...]
    h = (jnp.dot(x_ref[...][:, :8], wr_ref[...],
                 preferred_element_type=jnp.float32)
         + p[:, :8] / jnp.maximum(p[:, 8:9], 1.0) + br_ref[...])
    h_ref[...] = h
    part = jnp.concatenate([jnp.sum(h, axis=0, keepdims=True),
                            jnp.sum(h * h, axis=0, keepdims=True)], axis=1)

    @pl.when(i == 0)
    def _():
        acc[...] = jnp.zeros_like(acc)

    acc[...] += part

    @pl.when(i == pl.num_programs(0) - 1)
    def _():
        st_ref[...] = acc[...]


def _tc_finA(x16, p0, p1, wr, br):
    n = x16.shape[0]
    tn = _tn_for(n)
    grid = n // tn
    row = lambda i: (i, 0)
    zero = lambda i: (0, 0)
    return pl.pallas_call(
        _finA_body,
        grid=(grid,),
        in_specs=[
            pl.BlockSpec((tn, 16), row),
            pl.BlockSpec((tn, 16), row),
            pl.BlockSpec((tn, 16), row),
            pl.BlockSpec((8, 8), zero),
            pl.BlockSpec((1, 8), zero),
        ],
        out_specs=[pl.BlockSpec((tn, 8), row), pl.BlockSpec((1, 16), zero)],
        out_shape=[jax.ShapeDtypeStruct((n, 8), jnp.float32),
                   jax.ShapeDtypeStruct((1, 16), jnp.float32)],
        scratch_shapes=[pltpu.VMEM((1, 16), jnp.float32)],
    )(x16, p0, p1, wr, br)


def _finB1_body(h_ref, st_ref, g_ref, b_ref, out_ref):
    st = st_ref[...]
    n = pl.num_programs(0) * h_ref.shape[0]
    mu = st[:, :8] / n
    var = st[:, 8:] / n - mu * mu
    hn = (h_ref[...] - mu) * lax.rsqrt(var + _EPS) * g_ref[...] + b_ref[...]
    out_ref[...] = jnp.concatenate([hn, jnp.zeros_like(hn)], axis=1)


def _tc_finB1(h, st, g, b):
    n = h.shape[0]
    tn = _tn_for(n)
    grid = n // tn
    row = lambda i: (i, 0)
    zero = lambda i: (0, 0)
    return pl.pallas_call(
        _finB1_body,
        grid=(grid,),
        in_specs=[
            pl.BlockSpec((tn, 8), row),
            pl.BlockSpec((1, 16), zero),
            pl.BlockSpec((1, 8), zero),
            pl.BlockSpec((1, 8), zero),
        ],
        out_specs=pl.BlockSpec((tn, 16), row),
        out_shape=jax.ShapeDtypeStruct((n, 16), jnp.float32),
    )(h, st, g, b)


def _finB2_body(h_ref, st_ref, g_ref, b_ref, bv_ref, out_ref, acc):
    i = pl.program_id(0)
    st = st_ref[...]
    n = pl.num_programs(0) * h_ref.shape[0]
    mu = st[:, :8] / n
    var = st[:, 8:] / n - mu * mu
    hn = (h_ref[...] - mu) * lax.rsqrt(var + _EPS) * g_ref[...] + b_ref[...]
    onehot = (bv_ref[...] == lax.broadcasted_iota(jnp.int32, (1, 64), 1))
    onehot = onehot.astype(jnp.float32)
    hcat = jnp.concatenate([hn, jnp.ones((h_ref.shape[0], 1), jnp.float32),
                            jnp.zeros((h_ref.shape[0], 7), jnp.float32)], axis=1)
    part = lax.dot_general(onehot, hcat, (((0,), (0,)), ((), ())),
                           preferred_element_type=jnp.float32)

    @pl.when(i == 0)
    def _():
        acc[...] = jnp.zeros_like(acc)

    acc[...] += part

    @pl.when(i == pl.num_programs(0) - 1)
    def _():
        sums = acc[...][:, :8]
        means = sums / jnp.maximum(acc[...][:, 8:9], 1.0)
        out_ref[...] = jnp.concatenate([sums, means], axis=1)


def _tc_finB2(h, st, g, b, bv):
    n = h.shape[0]
    tn = _tn_for(n)
    grid = n // tn
    row = lambda i: (i, 0)
    zero = lambda i: (0, 0)
    return pl.pallas_call(
        _finB2_body,
        grid=(grid,),
        in_specs=[
            pl.BlockSpec((tn, 8), row),
            pl.BlockSpec((1, 16), zero),
            pl.BlockSpec((1, 8), zero),
            pl.BlockSpec((1, 8), zero),
            pl.BlockSpec((tn, 1), row),
        ],
        out_specs=pl.BlockSpec((64, 16), zero),
        out_shape=jax.ShapeDtypeStruct((64, 16), jnp.float32),
        scratch_shapes=[pltpu.VMEM((64, 16), jnp.float32)],
    )(h, st, g, b, bv)


def _np_consts():
    r = np.zeros((16, 64), np.float32)
    s = np.zeros((64, 16), np.float32)
    for i in range(8):
        r[i, 8 * i:8 * i + 8] = 1.0
        for o in range(8):
            s[8 * i + o, o] = 1.0
    eye8 = np.eye(8, dtype=np.float32)
    r_b = np.kron(eye8, r)
    s_b = np.kron(eye8, s)
    e8_b = np.zeros((1, 128), np.float32)
    e8_b[0, 8::16] = 1.0
    return jnp.asarray(r_b), jnp.asarray(s_b), jnp.asarray(e8_b)


def kernel(x, edge_index, edge_attr, batch_vector,
           W_nn1, b_nn1, W_root1, b_root1, gamma1, beta1,
           W_nn2, b_nn2, W_root2, b_root2, gamma2, beta2):
    n = x.shape[0]
    e = edge_index.shape[1]
    src_i = edge_index[0].astype(jnp.int32)
    dst_i = edge_index[1].astype(jnp.int32)
    bv = batch_vector.astype(jnp.int32).reshape(n, 1)
    x16 = jnp.concatenate([x, jnp.zeros((n, 8), jnp.float32)], axis=1)
    zeros16 = jnp.zeros((n, 16), jnp.float32)
    r_b, s_b, e8_b = _np_consts()
    ea_w = edge_attr.reshape(e // 8, 128)
    eye8 = jnp.eye(8, dtype=jnp.float32)
    wnn1_b = jnp.kron(eye8, W_nn1)
    wnn2_b = jnp.kron(eye8, W_nn2)
    bnn1_b = jnp.tile(b_nn1, 8).reshape(1, 512)
    bnn2_b = jnp.tile(b_nn2, 8).reshape(1, 512)
    br1 = b_root1.reshape(1, 8)
    br2 = b_root2.reshape(1, 8)
    g1 = gamma1.reshape(1, 8)
    g2 = gamma2.reshape(1, 8)
    be1 = beta1.reshape(1, 8)
    be2 = beta2.reshape(1, 8)

    xs1 = _sc_gather(x16, src_i)
    msg1 = _tc_msg(xs1.reshape(e // 8, 128), ea_w, wnn1_b, bnn1_b,
                   r_b, s_b, e8_b)
    parts1 = _sc_scatter(msg1.reshape(e, 16), dst_i, zeros16)
    h1, st1 = _tc_finA(x16, parts1[0], parts1[1], W_root1, br1)
    h16 = _tc_finB1(h1, st1, g1, be1)

    xs2 = _sc_gather(h16, src_i)
    msg2 = _tc_msg(xs2.reshape(e // 8, 128), ea_w, wnn2_b, bnn2_b,
                   r_b, s_b, e8_b)
    parts2 = _sc_scatter(msg2.reshape(e, 16), dst_i, zeros16)
    h2, st2 = _tc_finA(h16, parts2[0], parts2[1], W_root2, br2)
    return _tc_finB2(h2, st2, g2, be2, bv)


# speedup vs baseline: 1.2576x; 1.2576x over previous
import functools

import numpy as np
import jax
import jax.numpy as jnp
from jax import lax
from jax.experimental import pallas as pl
from jax.experimental.pallas import tpu as pltpu
from jax.experimental.pallas import tpu_sc as plsc

_EPS = 1e-5
_NC = 2
_NS = 16
_NW = _NC * _NS
_CH = 128


_KJ = 13


def _sc_gather(table16, idx1d):
    e = idx1d.shape[0]
    nch = e // _CH
    assert nch * _CH == e
    per = nch // _NW
    rem = nch - per * _NW
    ng = per // _KJ
    assert ng * _KJ == per, (nch, per)
    rows_n = _KJ * _CH
    mesh = plsc.VectorSubcoreMesh(core_axis_name="c", subcore_axis_name="s")

    @functools.partial(
        pl.kernel,
        out_type=jax.ShapeDtypeStruct((e, 16), jnp.float32),
        mesh=mesh,
        compiler_params=pltpu.CompilerParams(use_tc_tiling_on_sc=False),
        scratch_types=[
            pltpu.VMEM((rows_n,), jnp.int32),
            pltpu.VMEM((rows_n, 16), jnp.float32),
            pltpu.VMEM((_CH,), jnp.int32),
            pltpu.VMEM((_CH, 16), jnp.float32),
            pltpu.SemaphoreType.DMA,
        ],
    )
    def k(table_hbm, idx_hbm, out_hbm, idx_b, rows_v, idx_e, rows_e, sem):
        wid = lax.axis_index("s") * _NC + lax.axis_index("c")
        start = wid * per

        def group(g, carry):
            gc = start + g * _KJ
            pltpu.sync_copy(idx_hbm.at[pl.ds(gc * _CH, rows_n)], idx_b)
            descs = [
                pltpu.async_copy(table_hbm.at[idx_b.at[pl.ds(j * _CH, _CH)]],
                                 rows_v.at[pl.ds(j * _CH, _CH)], sem)
                for j in range(_KJ)
            ]
            for d in descs:
                d.wait()
            pltpu.sync_copy(rows_v, out_hbm.at[pl.ds(gc * _CH, rows_n)])
            return carry

        lax.fori_loop(0, ng, group, 0)

        @pl.when(wid < rem)
        def _():
            ec = _NW * per + wid
            pltpu.sync_copy(idx_hbm.at[pl.ds(ec * _CH, _CH)], idx_e)
            pltpu.async_copy(table_hbm.at[idx_e], rows_e, sem).wait()
            pltpu.sync_copy(rows_e, out_hbm.at[pl.ds(ec * _CH, _CH)])

    return k(table16, idx1d)


def _sc_scatter(msg16, idx1d, zeros16):
    e = idx1d.shape[0]
    n = zeros16.shape[0]
    nch = e // _CH
    per = nch // _NW
    rem = nch - per * _NW
    ng = per // _KJ
    assert ng * _KJ == per, (nch, per)
    rows_n = _KJ * _CH
    seg = n // _NS
    assert seg * _NS == n
    mesh = plsc.VectorSubcoreMesh(core_axis_name="c", subcore_axis_name="s")

    @functools.partial(
        pl.kernel,
        out_type=jax.ShapeDtypeStruct((_NC, n, 16), jnp.float32),
        mesh=mesh,
        compiler_params=pltpu.CompilerParams(use_tc_tiling_on_sc=False),
        scratch_types=[
            pltpu.VMEM((rows_n,), jnp.int32),
            pltpu.VMEM((rows_n, 16), jnp.float32),
            pltpu.VMEM((_CH,), jnp.int32),
            pltpu.VMEM((_CH, 16), jnp.float32),
            pltpu.VMEM_SHARED((n, 16), jnp.float32),
            pltpu.SemaphoreType.DMA,
        ],
    )
    def k(msg_hbm, idx_hbm, zeros_hbm, out_hbm, idx_b, rows_v, idx_e, rows_e,
          acc, sem):
        c = lax.axis_index("c")
        s = lax.axis_index("s")
        wid = s * _NC + c
        start = wid * per
        pltpu.sync_copy(zeros_hbm.at[pl.ds(s * seg, seg)],
                        acc.at[pl.ds(s * seg, seg)])
        plsc.subcore_barrier()

        def group(g, carry):
            gc = start + g * _KJ
            pltpu.sync_copy(idx_hbm.at[pl.ds(gc * _CH, rows_n)], idx_b)
            pltpu.sync_copy(msg_hbm.at[pl.ds(gc * _CH, rows_n)], rows_v)
            descs = [
                pltpu.async_copy(rows_v.at[pl.ds(j * _CH, _CH)],
                                 acc.at[idx_b.at[pl.ds(j * _CH, _CH)]],
                                 sem, add=True)
                for j in range(_KJ)
            ]
            for d in descs:
                d.wait()
            return carry

        lax.fori_loop(0, ng, group, 0)

        @pl.when(wid < rem)
        def _():
            ec = _NW * per + wid
            pltpu.sync_copy(idx_hbm.at[pl.ds(ec * _CH, _CH)], idx_e)
            pltpu.sync_copy(msg_hbm.at[pl.ds(ec * _CH, _CH)], rows_e)
            pltpu.sync_copy(rows_e, acc.at[idx_e], add=True)

        plsc.subcore_barrier()
        pltpu.sync_copy(acc.at[pl.ds(s * seg, seg)],
                        out_hbm.at[c, pl.ds(s * seg, seg)])

    return k(msg16, idx1d, zeros16)


def _msg_body(xs_ref, ea_ref, wnn_ref, bnn_ref, r_ref, s_ref, e8_ref, out_ref):
    m = jnp.tanh(jnp.dot(ea_ref[...], wnn_ref[...],
                         preferred_element_type=jnp.float32) + bnn_ref[...])
    xr = jnp.dot(xs_ref[...], r_ref[...], preferred_element_type=jnp.float32)
    msg = jnp.dot(xr * m, s_ref[...], preferred_element_type=jnp.float32)
    out_ref[...] = msg + e8_ref[...]


def _tc_msg(xs_w, ea_w, wnn_b, bnn_b, r_b, s_b, e8_b):
    e8r = xs_w.shape[0]
    te = 1000 if e8r % 1000 == 0 else e8r
    grid = e8r // te
    return pl.pallas_call(
        _msg_body,
        grid=(grid,),
        in_specs=[
            pl.BlockSpec((te, 128), lambda i: (i, 0)),
            pl.BlockSpec((te, 128), lambda i: (i, 0)),
            pl.BlockSpec((128, 512), lambda i: (0, 0)),
            pl.BlockSpec((1, 512), lambda i: (0, 0)),
            pl.BlockSpec((128, 512), lambda i: (0, 0)),
            pl.BlockSpec((512, 128), lambda i: (0, 0)),
            pl.BlockSpec((1, 128), lambda i: (0, 0)),
        ],
        out_specs=pl.BlockSpec((te, 128), lambda i: (i, 0)),
        out_shape=jax.ShapeDtypeStruct((e8r, 128), jnp.float32),
    )(xs_w, ea_w, wnn_b, bnn_b, r_b, s_b, e8_b)


def _tnw_for(nw):
    return nw


def _finA_body(x_ref, p0_ref, p1_ref, wrb_ref, brb_ref, cb_ref, ma_ref,
               gm_ref, h_ref, st_ref, acc):
    i = pl.program_id(0)
    p = p0_ref[0] + p1_ref[0]
    cnt = jnp.maximum(jnp.dot(p, cb_ref[...],
                              preferred_element_type=jnp.float32), 1.0)
    aggm = (p * ma_ref[...]) / cnt
    xr = jnp.dot(x_ref[...], wrb_ref[...], preferred_element_type=jnp.float32)
    h = xr + aggm + brb_ref[...]
    h_ref[...] = h
    hsum = jnp.sum(h, axis=0, keepdims=True)
    hsq = jnp.sum(h * h, axis=0, keepdims=True)
    part = jnp.concatenate(
        [jnp.dot(hsum, gm_ref[...], preferred_element_type=jnp.float32),
         jnp.dot(hsq, gm_ref[...], preferred_element_type=jnp.float32)],
        axis=1)

    @pl.when(i == 0)
    def _():
        acc[...] = jnp.zeros_like(acc)

    acc[...] += part

    @pl.when(i == pl.num_programs(0) - 1)
    def _():
        st_ref[...] = acc[...]


def _tc_finA(x_w, parts_w, wrb, brb, cb, ma, gm):
    nw = x_w.shape[0]
    tnw = _tnw_for(nw)
    grid = nw // tnw
    row = lambda i: (i, 0)
    zero = lambda i: (0, 0)
    return pl.pallas_call(
        _finA_body,
        grid=(grid,),
        in_specs=[
            pl.BlockSpec((tnw, 128), row),
            pl.BlockSpec((1, tnw, 128), lambda i: (0, i, 0)),
            pl.BlockSpec((1, tnw, 128), lambda i: (1, i, 0)),
            pl.BlockSpec((128, 128), zero),
            pl.BlockSpec((1, 128), zero),
            pl.BlockSpec((128, 128), zero),
            pl.BlockSpec((1, 128), zero),
            pl.BlockSpec((128, 16), zero),
        ],
        out_specs=[pl.BlockSpec((tnw, 128), row), pl.BlockSpec((1, 32), zero)],
        out_shape=[jax.ShapeDtypeStruct((nw, 128), jnp.float32),
                   jax.ShapeDtypeStruct((1, 32), jnp.float32)],
        scratch_shapes=[pltpu.VMEM((1, 32), jnp.float32)],
    )(x_w, parts_w, parts_w, wrb, brb, cb, ma, gm)


def _bn_wide(h, st, tm, gb, bb, n_nodes):
    mu16 = st[:, :16] * (1.0 / n_nodes)
    var16 = st[:, 16:] * (1.0 / n_nodes) - mu16 * mu16
    sc16 = lax.rsqrt(var16 + _EPS)
    mu_b = jnp.dot(mu16, tm, preferred_element_type=jnp.float32)
    sc_b = jnp.dot(sc16, tm, preferred_element_type=jnp.float32)
    return (h - mu_b) * sc_b * gb + bb


def _finB1_body(h_ref, st_ref, tm_ref, gb_ref, bb_ref, out_ref):
    n_nodes = pl.num_programs(0) * h_ref.shape[0] * 8
    out_ref[...] = _bn_wide(h_ref[...], st_ref[...], tm_ref[...],
                            gb_ref[...], bb_ref[...], n_nodes)


def _tc_finB1(h_w, st, tm, gb, bb):
    nw = h_w.shape[0]
    tnw = _tnw_for(nw)
    grid = nw // tnw
    row = lambda i: (i, 0)
    zero = lambda i: (0, 0)
    return pl.pallas_call(
        _finB1_body,
        grid=(grid,),
        in_specs=[
            pl.BlockSpec((tnw, 128), row),
            pl.BlockSpec((1, 32), zero),
            pl.BlockSpec((16, 128), zero),
            pl.BlockSpec((1, 128), zero),
            pl.BlockSpec((1, 128), zero),
        ],
        out_specs=pl.BlockSpec((tnw, 128), row),
        out_shape=jax.ShapeDtypeStruct((nw, 128), jnp.float32),
    )(h_w, st, tm, gb, bb)


def _finB2_body(h_ref, st_ref, tm_ref, gb_ref, bb_ref, e1_ref, bv_ref,
                out_ref, acc):
    i = pl.program_id(0)
    n_nodes = pl.num_programs(0) * h_ref.shape[0] * 8
    hn = _bn_wide(h_ref[...], st_ref[...], tm_ref[...],
                  gb_ref[...], bb_ref[...], n_nodes) + e1_ref[...]
    bv = bv_ref[...]
    iota64 = lax.broadcasted_iota(jnp.int32, (1, 64), 1)
    part = jnp.zeros((64, 16), jnp.float32)
    for k in range(8):
        hk = hn[:, 16 * k:16 * k + 16]
        oh = (bv[:, k:k + 1] == iota64).astype(jnp.float32)
        part = part + lax.dot_general(oh, hk, (((0,), (0,)), ((), ())),
                                      preferred_element_type=jnp.float32)

    @pl.when(i == 0)
    def _():
        acc[...] = jnp.zeros_like(acc)

    acc[...] += part

    @pl.when(i == pl.num_programs(0) - 1)
    def _():
        sums = acc[...][:, :8]
        means = sums / jnp.maximum(acc[...][:, 8:9], 1.0)
        out_ref[...] = jnp.concatenate([sums, means], axis=1)


def _tc_finB2(h_w, st, tm, gb, bb, e1, bv8):
    nw = h_w.shape[0]
    tnw = _tnw_for(nw)
    grid = nw // tnw
    row = lambda i: (i, 0)
    zero = lambda i: (0, 0)
    return pl.pallas_call(
        _finB2_body,
        grid=(grid,),
        in_specs=[
            pl.BlockSpec((tnw, 128), row),
            pl.BlockSpec((1, 32), zero),
            pl.BlockSpec((16, 128), zero),
            pl.BlockSpec((1, 128), zero),
            pl.BlockSpec((1, 128), zero),
            pl.BlockSpec((1, 128), zero),
            pl.BlockSpec((tnw, 8), row),
        ],
        out_specs=pl.BlockSpec((64, 16), zero),
        out_shape=jax.ShapeDtypeStruct((64, 16), jnp.float32),
        scratch_shapes=[pltpu.VMEM((64, 16), jnp.float32)],
    )(h_w, st, tm, gb, bb, e1, bv8)


def _np_consts():
    r = np.zeros((16, 64), np.float32)
    s = np.zeros((64, 16), np.float32)
    for i in range(8):
        r[i, 8 * i:8 * i + 8] = 1.0
        for o in range(8):
            s[8 * i + o, o] = 1.0
    eye8 = np.eye(8, dtype=np.float32)
    r_b = np.kron(eye8, r)
    s_b = np.kron(eye8, s)
    e8_b = np.zeros((1, 128), np.float32)
    e8_b[0, 8::16] = 1.0
    cb = np.zeros((128, 128), np.float32)
    ma = np.zeros((1, 128), np.float32)
    for k in range(8):
        for o in range(8):
            cb[16 * k + 8, 16 * k + o] = 1.0
            ma[0, 16 * k + o] = 1.0
    gm = np.tile(np.eye(16, dtype=np.float32), (8, 1))
    tm = np.tile(np.eye(16, dtype=np.float32), (1, 8))
    return tuple(jnp.asarray(a) for a in (r_b, s_b, e8_b, cb, ma, gm, tm))


def _wide8(v):
    return jnp.tile(jnp.concatenate([v, jnp.zeros((8,), jnp.float32)]),
                    8).reshape(1, 128)


def kernel(x, edge_index, edge_attr, batch_vector,
           W_nn1, b_nn1, W_root1, b_root1, gamma1, beta1,
           W_nn2, b_nn2, W_root2, b_root2, gamma2, beta2):
    n = x.shape[0]
    e = edge_index.shape[1]
    nw = n // 8
    src_i = edge_index[0].astype(jnp.int32)
    dst_i = edge_index[1].astype(jnp.int32)
    bv8 = batch_vector.astype(jnp.int32).reshape(nw, 8)
    x_w = jnp.concatenate([x, jnp.zeros((n, 8), jnp.float32)],
                          axis=1).reshape(nw, 128)
    zeros16 = jnp.zeros((n, 16), jnp.float32)
    r_b, s_b, e8_b, cb, ma, gm, tm = _np_consts()
    ea_w = edge_attr.reshape(e // 8, 128)
    eye8 = jnp.eye(8, dtype=jnp.float32)
    wnn1_b = jnp.kron(eye8, W_nn1)
    wnn2_b = jnp.kron(eye8, W_nn2)
    bnn1_b = jnp.tile(b_nn1, 8).reshape(1, 512)
    bnn2_b = jnp.tile(b_nn2, 8).reshape(1, 512)
    pad16 = lambda w: jnp.pad(w, ((0, 8), (0, 8)))
    wrb1 = jnp.kron(eye8, pad16(W_root1))
    wrb2 = jnp.kron(eye8, pad16(W_root2))
    brb1 = _wide8(b_root1)
    brb2 = _wide8(b_root2)
    gb1 = _wide8(gamma1)
    gb2 = _wide8(gamma2)
    bb1 = _wide8(beta1)
    bb2 = _wide8(beta2)

    xs1 = _sc_gather(x_w.reshape(n, 16), src_i)
    msg1 = _tc_msg(xs1.reshape(e // 8, 128), ea_w, wnn1_b, bnn1_b,
                   r_b, s_b, e8_b)
    parts1 = _sc_scatter(msg1.reshape(e, 16), dst_i, zeros16)
    h1_w, st1 = _tc_finA(x_w, parts1.reshape(2, nw, 128), wrb1, brb1,
                         cb, ma, gm)
    h16_w = _tc_finB1(h1_w, st1, tm, gb1, bb1)

    xs2 = _sc_gather(h16_w.reshape(n, 16), src_i)
    msg2 = _tc_msg(xs2.reshape(e // 8, 128), ea_w, wnn2_b, bnn2_b,
                   r_b, s_b, e8_b)
    parts2 = _sc_scatter(msg2.reshape(e, 16), dst_i, zeros16)
    h2_w, st2 = _tc_finA(h16_w, parts2.reshape(2, nw, 128), wrb2, brb2,
                         cb, ma, gm)
    return _tc_finB2(h2_w, st2, tm, gb2, bb2, e8_b, bv8)
